# Initial kernel scaffold; baseline (speedup 1.0000x reference)
import jax
import jax.numpy as jnp
from jax.experimental import pallas as pl


def kernel(x, time, token_table, time_table):
    raise NotImplementedError("write your pallas kernel here")


# speedup vs baseline: 1.8072x; 1.8072x over previous
import functools
import math

import jax
import jax.numpy as jnp
from jax import lax
from jax.experimental import pallas as pl
from jax.experimental.pallas import tpu as pltpu
from jax.experimental.pallas import tpu_sc as plsc

D = 64
CHUNK = 128
NC, NS = 2, 16
NW = NC * NS

_GDN = lax.GatherDimensionNumbers(
    offset_dims=(), collapsed_slice_dims=(0,), start_index_map=(0,))


def _bcast_lane(vec, r):
    idx = jnp.full((16, 1), r, jnp.int32)
    return lax.gather(vec, idx, dimension_numbers=_GDN, slice_sizes=(1,),
                      mode=lax.GatherScatterMode.PROMISE_IN_BOUNDS)


def _body(x_hbm, t_hbm, tok_hbm, tim_hbm, out_hbm,
          xidx_v, tidx_v, tim8_v, rows_v, outb_v, sem_g, sem_o, sem_i):
    wid = lax.axis_index("c") * NS + lax.axis_index("s")
    steps = x_hbm.shape[1]

    pltpu.async_copy(x_hbm.at[wid], xidx_v, sem_i).wait()
    pltpu.async_copy(t_hbm.at[wid], tidx_v, sem_i).wait()

    pltpu.async_copy(tim_hbm, tim8_v, sem_i).wait()
    scale = jnp.float32(math.sqrt(float(D)))

    def _prep(i, _):
        s = jnp.where(i == 0, jnp.float32(0.0), scale)
        for c in range(D // 16):
            off = i * D + 16 * c
            tim8_v[pl.ds(off, 16)] = tim8_v[pl.ds(off, 16)] * s
        return 0

    lax.fori_loop(0, tim_hbm.shape[0] // D, _prep, 0, unroll=False)

    def _step(s, _):
        pltpu.async_copy(tok_hbm.at[xidx_v.at[s]], rows_v, sem_g).wait()

        def _group(g, _):
            xv = xidx_v[s, pl.ds(g * 16, 16)]
            tv = tidx_v[s, pl.ds(g * 16, 16)]
            sx = jnp.where(xv != 0, scale, jnp.float32(0.0))
            for r in range(16):
                bx = _bcast_lane(sx, r)
                tb = _bcast_lane(tv, r) * D
                row = g * 16 + r
                for c in range(D // 16):
                    cidx = lax.iota(jnp.int32, 16) + 16 * c
                    tok = rows_v[row, pl.ds(16 * c, 16)]
                    tim = plsc.load_gather(tim8_v, [tb + cidx])
                    outb_v[row, pl.ds(16 * c, 16)] = tok * bx + tim
            return 0

        lax.fori_loop(0, CHUNK // 16, _group, 0, unroll=False)

        base = (wid * steps + s) * CHUNK
        pltpu.async_copy(outb_v, out_hbm.at[pl.ds(base, CHUNK)], sem_o).wait()
        return 0

    lax.fori_loop(0, steps, _step, 0, unroll=False)


@jax.jit
def kernel(x, time, token_table, time_table):
    B, L = x.shape
    N = B * L
    steps = N // (NW * CHUNK)
    x3 = x.reshape(NW, steps, CHUNK).astype(jnp.int32)
    t3 = time.reshape(NW, steps, CHUNK).astype(jnp.int32)

    mesh = plsc.VectorSubcoreMesh(core_axis_name="c", subcore_axis_name="s")
    run = pl.kernel(
        _body,
        mesh=mesh,
        compiler_params=pltpu.CompilerParams(
            needs_layout_passes=False, use_tc_tiling_on_sc=False),
        out_type=jax.ShapeDtypeStruct((N, D), jnp.float32),
        scratch_types=[
            pltpu.VMEM((steps, CHUNK), jnp.int32),
            pltpu.VMEM((steps, CHUNK), jnp.int32),
            pltpu.VMEM((time_table.shape[0] * D,), jnp.float32),
            pltpu.VMEM((CHUNK, D), jnp.float32),
            pltpu.VMEM((CHUNK, D), jnp.float32),
            pltpu.SemaphoreType.DMA,
            pltpu.SemaphoreType.DMA,
            pltpu.SemaphoreType.DMA,
        ],
    )
    out = run(x3, t3, token_table, time_table.reshape(-1))
    return out.reshape(B, L, D)


# speedup vs baseline: 2.0920x; 1.1576x over previous
import functools
import math

import jax
import jax.numpy as jnp
from jax import lax
from jax.experimental import pallas as pl
from jax.experimental.pallas import tpu as pltpu
from jax.experimental.pallas import tpu_sc as plsc

D = 64
CHUNK = 128
NBUF = 4
NC, NS = 2, 16
NW = NC * NS

_GDN = lax.GatherDimensionNumbers(
    offset_dims=(), collapsed_slice_dims=(0,), start_index_map=(0,))


def _bcast_lane(vec, r):
    idx = jnp.full((16, 1), r, jnp.int32)
    return lax.gather(vec, idx, dimension_numbers=_GDN, slice_sizes=(1,),
                      mode=lax.GatherScatterMode.PROMISE_IN_BOUNDS)


def _body(x_hbm, t_hbm, tok_hbm, tim_hbm, out_hbm,
          xidx_v, tidx_v, tim8_v, gbufs, obufs, gsems, osems, sem_i):
    wid = lax.axis_index("c") * NS + lax.axis_index("s")
    steps = x_hbm.shape[1]
    outer = steps // NBUF

    pltpu.async_copy(x_hbm.at[wid], xidx_v, sem_i).wait()
    pltpu.async_copy(t_hbm.at[wid], tidx_v, sem_i).wait()

    pltpu.async_copy(tim_hbm, tim8_v, sem_i).wait()
    scale = jnp.float32(math.sqrt(float(D)))

    def _prep(i, _):
        s = jnp.where(i == 0, jnp.float32(0.0), scale)
        for c in range(D // 16):
            off = i * D + 16 * c
            tim8_v[pl.ds(off, 16)] = tim8_v[pl.ds(off, 16)] * s
        return 0

    lax.fori_loop(0, tim_hbm.shape[0] // D, _prep, 0, unroll=False)

    def _start_gather(s, b):
        return pltpu.async_copy(tok_hbm.at[xidx_v.at[s]], gbufs[b], gsems[b])

    def _compute(s, b):
        def _group(g, _):
            xv = xidx_v[s, pl.ds(g * 16, 16)]
            tv = tidx_v[s, pl.ds(g * 16, 16)]
            sx = jnp.where(xv != 0, scale, jnp.float32(0.0))
            for r in range(16):
                bx = _bcast_lane(sx, r)
                tb = _bcast_lane(tv, r) * D
                row = g * 16 + r
                for c in range(D // 16):
                    cidx = lax.iota(jnp.int32, 16) + 16 * c
                    tok = gbufs[b][row, pl.ds(16 * c, 16)]
                    tim = plsc.load_gather(tim8_v, [tb + cidx])
                    obufs[b][row, pl.ds(16 * c, 16)] = tok * bx + tim
            return 0

        lax.fori_loop(0, CHUNK // 16, _group, 0, unroll=False)

    for b in range(NBUF):
        _start_gather(b, b)

    def _outer(i, _):
        for b in range(NBUF):
            s = i * NBUF + b
            pltpu.make_async_copy(tok_hbm.at[xidx_v.at[s]], gbufs[b],
                                  gsems[b]).wait()

            @pl.when(i > 0)
            def _():
                base0 = (wid * steps + (s - NBUF)) * CHUNK
                pltpu.make_async_copy(
                    obufs[b], out_hbm.at[pl.ds(base0, CHUNK)], osems[b]).wait()

            _compute(s, b)
            base = (wid * steps + s) * CHUNK
            pltpu.async_copy(obufs[b], out_hbm.at[pl.ds(base, CHUNK)],
                             osems[b])

            @pl.when(i < outer - 1)
            def _():
                _start_gather(s + NBUF, b)
        return 0

    lax.fori_loop(0, outer, _outer, 0, unroll=False)

    for b in range(NBUF):
        s = (outer - 1) * NBUF + b
        base = (wid * steps + s) * CHUNK
        pltpu.make_async_copy(obufs[b], out_hbm.at[pl.ds(base, CHUNK)],
                              osems[b]).wait()


@jax.jit
def kernel(x, time, token_table, time_table):
    B, L = x.shape
    N = B * L
    steps = N // (NW * CHUNK)
    x3 = x.reshape(NW, steps, CHUNK).astype(jnp.int32)
    t3 = time.reshape(NW, steps, CHUNK).astype(jnp.int32)

    mesh = plsc.VectorSubcoreMesh(core_axis_name="c", subcore_axis_name="s")
    run = pl.kernel(
        _body,
        mesh=mesh,
        compiler_params=pltpu.CompilerParams(
            needs_layout_passes=False, use_tc_tiling_on_sc=False),
        out_type=jax.ShapeDtypeStruct((N, D), jnp.float32),
        scratch_types=[
            pltpu.VMEM((steps, CHUNK), jnp.int32),
            pltpu.VMEM((steps, CHUNK), jnp.int32),
            pltpu.VMEM((time_table.shape[0] * D,), jnp.float32),
            [pltpu.VMEM((CHUNK, D), jnp.float32) for _ in range(NBUF)],
            [pltpu.VMEM((CHUNK, D), jnp.float32) for _ in range(NBUF)],
            [pltpu.SemaphoreType.DMA for _ in range(NBUF)],
            [pltpu.SemaphoreType.DMA for _ in range(NBUF)],
            pltpu.SemaphoreType.DMA,
        ],
    )
    out = run(x3, t3, token_table, time_table.reshape(-1))
    return out.reshape(B, L, D)
